# baseline (speedup 1.0000x reference)
import jax
import jax.numpy as jnp
from jax.experimental import pallas as pl

B, H, W = 4, 640, 640
N_PRED, NC, NT = 20000, 20, 200
CONF_THRES, IOU_THRES = 0.25, 0.45
MAX_DET = 300


def _xywh2xyxy(x):
    xy = x[..., :2]
    wh = x[..., 2:4]
    return jnp.concatenate([xy - wh / 2.0, xy + wh / 2.0], axis=-1)


def _box_iou(a, b):
    area_a = jnp.clip(a[:, 2] - a[:, 0], 0) * jnp.clip(a[:, 3] - a[:, 1], 0)
    area_b = jnp.clip(b[:, 2] - b[:, 0], 0) * jnp.clip(b[:, 3] - b[:, 1], 0)
    lt = jnp.maximum(a[:, None, :2], b[None, :, :2])
    rb = jnp.minimum(a[:, None, 2:4], b[None, :, 2:4])
    wh = jnp.clip(rb - lt, 0)
    inter = wh[..., 0] * wh[..., 1]
    return inter / (area_a[:, None] + area_b[None, :] - inter + 1e-7)


def _conf_kernel(o_ref, conf_ref):
    conf_ref[...] = o_ref[..., 4:5] * o_ref[..., 5:]


def _nms_single(pred):
    boxes = _xywh2xyxy(pred[:, :4])
    conf = pl.pallas_call(
        _conf_kernel,
        out_shape=jax.ShapeDtypeStruct((N_PRED, NC), jnp.float32),
    )(pred)
    best_conf = jnp.max(conf, axis=1)
    best_cls = jnp.argmax(conf, axis=1).astype(jnp.float32)
    score = jnp.where(best_conf > CONF_THRES, best_conf, -1.0)
    top_s, top_i = jax.lax.top_k(score, MAX_DET)
    pb = boxes[top_i]
    pc = best_cls[top_i]
    cb = pb + pc[:, None] * 4096.0
    iou = _box_iou(cb, cb)
    keep0 = top_s > CONF_THRES
    idx = jnp.arange(MAX_DET)

    def body(i, keep):
        sup = (iou[i] > IOU_THRES) & (idx > i) & keep[i]
        return keep & (~sup)

    keep = jax.lax.fori_loop(0, MAX_DET, body, keep0)
    return pb, top_s, pc, keep


def kernel(imgs, targets, out, shapes):
    height, width = imgs.shape[2], imgs.shape[3]
    iouv = jnp.linspace(0.5, 0.95, 10)
    gain = jnp.array([width, height, width, height], dtype=jnp.float32)
    tbox = _xywh2xyxy(targets[:, 2:6] * gain)
    tcls = targets[:, 1]
    timg = targets[:, 0]
    corrects, confs, pclss, valids = [], [], [], []
    for si in range(out.shape[0]):
        pb, psc, pc, keep = _nms_single(out[si])
        lmask = (timg == si)
        m = _box_iou(pb, tbox) * (pc[:, None] == tcls[None, :]) * lmask[None, :]
        best_iou = jnp.max(m, axis=1)
        correct = (best_iou[:, None] > iouv[None, :]) & keep[:, None]
        corrects.append(correct)
        confs.append(jnp.where(keep, psc, -1.0))
        pclss.append(pc)
        valids.append(keep)
    correct = jnp.concatenate(corrects, 0)
    conf = jnp.concatenate(confs, 0)
    pcls = jnp.concatenate(pclss, 0)
    valid = jnp.concatenate(valids, 0)
    order = jnp.argsort(-conf)
    correct = correct[order]
    pcls = pcls[order]
    valid = valid[order]
    n_gt = jnp.sum(tcls[None, :] == jnp.arange(NC, dtype=jnp.float32)[:, None], axis=1)
    eps = 1e-9
    x101 = jnp.linspace(0.0, 1.0, 101)

    def ap_one(rec, prec):
        mrec = jnp.concatenate([jnp.array([0.0]), rec, jnp.array([1.0])])
        mpre = jnp.concatenate([jnp.array([1.0]), prec, jnp.array([0.0])])
        mpre = jnp.flip(jax.lax.cummax(jnp.flip(mpre)))
        interp = jnp.interp(x101, mrec, mpre)
        return jnp.sum((interp[1:] + interp[:-1]) * 0.5 * (x101[1:] - x101[:-1]))

    ps, rs, ap50s, aps = [], [], [], []
    for c in range(NC):
        mask = (pcls == c) & valid
        tp = (correct & mask[:, None]).astype(jnp.float32)
        fp = ((~correct) & mask[:, None]).astype(jnp.float32)
        tpc = jnp.cumsum(tp, axis=0)
        fpc = jnp.cumsum(fp, axis=0)
        recall = tpc / (n_gt[c] + eps)
        precision = tpc / (tpc + fpc + eps)
        ap_c = jax.vmap(ap_one, in_axes=(1, 1))(recall, precision)
        f1 = 2 * precision[:, 0] * recall[:, 0] / (precision[:, 0] + recall[:, 0] + eps)
        bi = jnp.argmax(f1)
        ps.append(precision[bi, 0])
        rs.append(recall[bi, 0])
        ap50s.append(ap_c[0])
        aps.append(jnp.mean(ap_c))
    p = jnp.stack(ps)
    r = jnp.stack(rs)
    ap50 = jnp.stack(ap50s)
    ap = jnp.stack(aps)
    has = (n_gt > 0).astype(jnp.float32)
    denom = jnp.maximum(jnp.sum(has), 1.0)
    mp = jnp.sum(p * has) / denom
    mr = jnp.sum(r * has) / denom
    map50 = jnp.sum(ap50 * has) / denom
    mapv = jnp.sum(ap * has) / denom
    fi = 0.1 * map50 + 0.9 * mapv
    return jnp.stack([fi, mp, mr, map50, mapv])


# speedup vs baseline: 1.3467x; 1.3467x over previous
import jax
import jax.numpy as jnp
from jax.experimental import pallas as pl
from jax.experimental.pallas import tpu as pltpu

B, H, W = 4, 640, 640
N_PRED, NC, NT = 20000, 20, 200
CONF_THRES, IOU_THRES = 0.25, 0.45
MAX_DET = 300


def _score_kernel(o_ref, score_ref, cls_ref):
    o = o_ref[0]
    conf = o[4:5, :] * o[5:5 + NC, :]
    best = jnp.max(conf, axis=0, keepdims=True)
    sub = jax.lax.broadcasted_iota(jnp.int32, conf.shape, 0)
    cls = jnp.min(jnp.where(conf >= best, sub, NC), axis=0,
                  keepdims=True)
    score_ref[0] = jnp.where(best > CONF_THRES, best, -1.0)
    cls_ref[0] = cls.astype(jnp.float32)


def _nms_kernel(p_ref, pt_ref, pcc_ref, pcr_ref, ts_ref, tgt_ref, iouv_ref,
                keep_ref, conf_ref, cor_ref, iou_scr):
    si = jax.lax.convert_element_type(pl.program_id(0), jnp.float32)
    p = p_ref[0]
    pt = pt_ref[0]
    pcc = pcc_ref[0]
    pcr = pcr_ref[0]
    ts = ts_ref[0]
    tgt = tgt_ref[...]
    iouv = iouv_ref[...]

    xc, yc, wc, hc = p[:, 0:1], p[:, 1:2], p[:, 2:3], p[:, 3:4]
    x1c, y1c = xc - wc / 2.0, yc - hc / 2.0
    x2c, y2c = xc + wc / 2.0, yc + hc / 2.0
    xr, yr, wr, hr = pt[0:1, :], pt[1:2, :], pt[2:3, :], pt[3:4, :]
    x1r, y1r = xr - wr / 2.0, yr - hr / 2.0
    x2r, y2r = xr + wr / 2.0, yr + hr / 2.0

    offc = pcc * 4096.0
    offr = pcr * 4096.0
    cx1c, cy1c, cx2c, cy2c = x1c + offc, y1c + offc, x2c + offc, y2c + offc
    cx1r, cy1r, cx2r, cy2r = x1r + offr, y1r + offr, x2r + offr, y2r + offr

    area_c = jnp.maximum(cx2c - cx1c, 0.0) * jnp.maximum(cy2c - cy1c, 0.0)
    area_r = jnp.maximum(cx2r - cx1r, 0.0) * jnp.maximum(cy2r - cy1r, 0.0)
    w = jnp.maximum(jnp.minimum(cx2c, cx2r) - jnp.maximum(cx1c, cx1r), 0.0)
    h = jnp.maximum(jnp.minimum(cy2c, cy2r) - jnp.maximum(cy1c, cy1r), 0.0)
    inter = w * h
    iou_scr[...] = inter / (area_c + area_r - inter + 1e-7)

    lane = jax.lax.broadcasted_iota(jnp.int32, (1, MAX_DET), 1)
    keep0 = jnp.where(ts > CONF_THRES, 1.0, 0.0)

    def body(i, keep):
        row = iou_scr[pl.ds(i, 1), :]
        ki = jnp.sum(jnp.where(lane == i, keep, 0.0))
        sup = jnp.where((row > IOU_THRES) & (lane > i), ki, 0.0)
        return keep * (1.0 - sup)

    keep = jax.lax.fori_loop(0, MAX_DET, body, keep0)

    keep_ref[0] = keep
    conf_ref[0] = jnp.where(keep > 0.0, ts, -1.0)

    tx, ty = tgt[:, 2:3] * float(W), tgt[:, 3:4] * float(H)
    tw, th = tgt[:, 4:5] * float(W), tgt[:, 5:6] * float(H)
    tx1, ty1 = tx - tw / 2.0, ty - th / 2.0
    tx2, ty2 = tx + tw / 2.0, ty + th / 2.0
    t_area = jnp.maximum(tx2 - tx1, 0.0) * jnp.maximum(ty2 - ty1, 0.0)
    p_area = jnp.maximum(x2r - x1r, 0.0) * jnp.maximum(y2r - y1r, 0.0)
    w2 = jnp.maximum(jnp.minimum(tx2, x2r) - jnp.maximum(tx1, x1r), 0.0)
    h2 = jnp.maximum(jnp.minimum(ty2, y2r) - jnp.maximum(ty1, y1r), 0.0)
    inter2 = w2 * h2
    iou2 = inter2 / (t_area + p_area - inter2 + 1e-7)
    clsm = jnp.where(tgt[:, 1:2] == pcr, 1.0, 0.0)
    imgm = jnp.where(tgt[:, 0:1] == si, 1.0, 0.0)
    m = iou2 * clsm * imgm
    best = jnp.max(m, axis=0, keepdims=True)
    cor_ref[0] = jnp.where((best > iouv) & (keep > 0.0), 1.0, 0.0)


def kernel(imgs, targets, out, shapes):
    iouv = jnp.linspace(0.5, 0.95, 10)
    iouv_col = iouv.reshape(10, 1)

    outT = jnp.swapaxes(out, 1, 2)
    score, cls = pl.pallas_call(
        _score_kernel,
        grid=(B,),
        in_specs=[pl.BlockSpec((1, 5 + NC, N_PRED), lambda i: (i, 0, 0))],
        out_specs=[pl.BlockSpec((1, 1, N_PRED), lambda i: (i, 0, 0)),
                   pl.BlockSpec((1, 1, N_PRED), lambda i: (i, 0, 0))],
        out_shape=[jax.ShapeDtypeStruct((B, 1, N_PRED), jnp.float32),
                   jax.ShapeDtypeStruct((B, 1, N_PRED), jnp.float32)],
    )(outT)
    score = score[:, 0, :]
    cls = cls[:, 0, :]

    top_s, top_i = jax.lax.top_k(score, MAX_DET)
    pxywh = jnp.take_along_axis(out[:, :, :4], top_i[..., None], axis=1)
    pc = jnp.take_along_axis(cls, top_i, axis=1)
    pt = jnp.swapaxes(pxywh, 1, 2)

    keep, confo, cor = pl.pallas_call(
        _nms_kernel,
        grid=(B,),
        in_specs=[
            pl.BlockSpec((1, MAX_DET, 4), lambda i: (i, 0, 0)),
            pl.BlockSpec((1, 4, MAX_DET), lambda i: (i, 0, 0)),
            pl.BlockSpec((1, MAX_DET, 1), lambda i: (i, 0, 0)),
            pl.BlockSpec((1, 1, MAX_DET), lambda i: (i, 0, 0)),
            pl.BlockSpec((1, 1, MAX_DET), lambda i: (i, 0, 0)),
            pl.BlockSpec((NT, 6), lambda i: (0, 0)),
            pl.BlockSpec((10, 1), lambda i: (0, 0)),
        ],
        out_specs=[
            pl.BlockSpec((1, 1, MAX_DET), lambda i: (i, 0, 0)),
            pl.BlockSpec((1, 1, MAX_DET), lambda i: (i, 0, 0)),
            pl.BlockSpec((1, 10, MAX_DET), lambda i: (i, 0, 0)),
        ],
        out_shape=[
            jax.ShapeDtypeStruct((B, 1, MAX_DET), jnp.float32),
            jax.ShapeDtypeStruct((B, 1, MAX_DET), jnp.float32),
            jax.ShapeDtypeStruct((B, 10, MAX_DET), jnp.float32),
        ],
        scratch_shapes=[pltpu.VMEM((MAX_DET, MAX_DET), jnp.float32)],
    )(pxywh, pt, pc[..., None], pc[:, None, :], top_s[:, None, :],
      targets, iouv_col)

    valid = keep.reshape(B * MAX_DET) > 0.0
    conf = confo.reshape(B * MAX_DET)
    pcls = pc.reshape(B * MAX_DET)
    correct = jnp.swapaxes(cor, 1, 2).reshape(B * MAX_DET, 10) > 0.0

    order = jnp.argsort(-conf)
    correct = correct[order]
    pcls = pcls[order]
    valid = valid[order]

    tcls = targets[:, 1]
    n_gt = jnp.sum(tcls[None, :] == jnp.arange(NC, dtype=jnp.float32)[:, None],
                   axis=1)
    eps = 1e-9
    x101 = jnp.linspace(0.0, 1.0, 101)

    def ap_one(rec, prec):
        mrec = jnp.concatenate([jnp.array([0.0]), rec, jnp.array([1.0])])
        mpre = jnp.concatenate([jnp.array([1.0]), prec, jnp.array([0.0])])
        mpre = jnp.flip(jax.lax.cummax(jnp.flip(mpre)))
        interp = jnp.interp(x101, mrec, mpre)
        return jnp.sum((interp[1:] + interp[:-1]) * 0.5 * (x101[1:] - x101[:-1]))

    ps, rs, ap50s, aps = [], [], [], []
    for c in range(NC):
        mask = (pcls == c) & valid
        tp = (correct & mask[:, None]).astype(jnp.float32)
        fp = ((~correct) & mask[:, None]).astype(jnp.float32)
        tpc = jnp.cumsum(tp, axis=0)
        fpc = jnp.cumsum(fp, axis=0)
        recall = tpc / (n_gt[c] + eps)
        precision = tpc / (tpc + fpc + eps)
        ap_c = jax.vmap(ap_one, in_axes=(1, 1))(recall, precision)
        f1 = 2 * precision[:, 0] * recall[:, 0] / (precision[:, 0] + recall[:, 0] + eps)
        bi = jnp.argmax(f1)
        ps.append(precision[bi, 0])
        rs.append(recall[bi, 0])
        ap50s.append(ap_c[0])
        aps.append(jnp.mean(ap_c))
    p = jnp.stack(ps)
    r = jnp.stack(rs)
    ap50 = jnp.stack(ap50s)
    ap = jnp.stack(aps)
    has = (n_gt > 0).astype(jnp.float32)
    denom = jnp.maximum(jnp.sum(has), 1.0)
    mp = jnp.sum(p * has) / denom
    mr = jnp.sum(r * has) / denom
    map50 = jnp.sum(ap50 * has) / denom
    mapv = jnp.sum(ap * has) / denom
    fi = 0.1 * map50 + 0.9 * mapv
    return jnp.stack([fi, mp, mr, map50, mapv])


# speedup vs baseline: 13.4563x; 9.9922x over previous
import jax
import jax.numpy as jnp
from jax.experimental import pallas as pl
from jax.experimental.pallas import tpu as pltpu

B, H, W = 4, 640, 640
N_PRED, NC, NT = 20000, 20, 200
CONF_THRES, IOU_THRES = 0.25, 0.45
MAX_DET = 300


def _score_kernel(o_ref, score_ref, cls_ref):
    o = o_ref[0]
    conf = o[4:5, :] * o[5:5 + NC, :]
    best = jnp.max(conf, axis=0, keepdims=True)
    sub = jax.lax.broadcasted_iota(jnp.int32, conf.shape, 0)
    cls = jnp.min(jnp.where(conf >= best, sub, NC), axis=0,
                  keepdims=True)
    score_ref[0] = jnp.where(best > CONF_THRES, best, -1.0)
    cls_ref[0] = cls.astype(jnp.float32)


def _nms_kernel(p_ref, pt_ref, pcc_ref, pcr_ref, ts_ref, tgt_ref, iouv_ref,
                keep_ref, conf_ref, cor_ref, iou_scr):
    si = jax.lax.convert_element_type(pl.program_id(0), jnp.float32)
    p = p_ref[0]
    pt = pt_ref[0]
    pcc = pcc_ref[0]
    pcr = pcr_ref[0]
    ts = ts_ref[0]
    tgt = tgt_ref[...]
    iouv = iouv_ref[...]

    xc, yc, wc, hc = p[:, 0:1], p[:, 1:2], p[:, 2:3], p[:, 3:4]
    x1c, y1c = xc - wc / 2.0, yc - hc / 2.0
    x2c, y2c = xc + wc / 2.0, yc + hc / 2.0
    xr, yr, wr, hr = pt[0:1, :], pt[1:2, :], pt[2:3, :], pt[3:4, :]
    x1r, y1r = xr - wr / 2.0, yr - hr / 2.0
    x2r, y2r = xr + wr / 2.0, yr + hr / 2.0

    offc = pcc * 4096.0
    offr = pcr * 4096.0
    cx1c, cy1c, cx2c, cy2c = x1c + offc, y1c + offc, x2c + offc, y2c + offc
    cx1r, cy1r, cx2r, cy2r = x1r + offr, y1r + offr, x2r + offr, y2r + offr

    area_c = jnp.maximum(cx2c - cx1c, 0.0) * jnp.maximum(cy2c - cy1c, 0.0)
    area_r = jnp.maximum(cx2r - cx1r, 0.0) * jnp.maximum(cy2r - cy1r, 0.0)
    w = jnp.maximum(jnp.minimum(cx2c, cx2r) - jnp.maximum(cx1c, cx1r), 0.0)
    h = jnp.maximum(jnp.minimum(cy2c, cy2r) - jnp.maximum(cy1c, cy1r), 0.0)
    inter = w * h
    iou_scr[...] = inter / (area_c + area_r - inter + 1e-7)

    lane = jax.lax.broadcasted_iota(jnp.int32, (1, MAX_DET), 1)
    keep0 = jnp.where(ts > CONF_THRES, 1.0, 0.0)

    def body(i, keep):
        row = iou_scr[pl.ds(i, 1), :]
        ki = jnp.sum(jnp.where(lane == i, keep, 0.0))
        sup = jnp.where((row > IOU_THRES) & (lane > i), ki, 0.0)
        return keep * (1.0 - sup)

    keep = jax.lax.fori_loop(0, MAX_DET, body, keep0)

    keep_ref[0] = keep
    conf_ref[0] = jnp.where(keep > 0.0, ts, -1.0)

    tx, ty = tgt[:, 2:3] * float(W), tgt[:, 3:4] * float(H)
    tw, th = tgt[:, 4:5] * float(W), tgt[:, 5:6] * float(H)
    tx1, ty1 = tx - tw / 2.0, ty - th / 2.0
    tx2, ty2 = tx + tw / 2.0, ty + th / 2.0
    t_area = jnp.maximum(tx2 - tx1, 0.0) * jnp.maximum(ty2 - ty1, 0.0)
    p_area = jnp.maximum(x2r - x1r, 0.0) * jnp.maximum(y2r - y1r, 0.0)
    w2 = jnp.maximum(jnp.minimum(tx2, x2r) - jnp.maximum(tx1, x1r), 0.0)
    h2 = jnp.maximum(jnp.minimum(ty2, y2r) - jnp.maximum(ty1, y1r), 0.0)
    inter2 = w2 * h2
    iou2 = inter2 / (t_area + p_area - inter2 + 1e-7)
    clsm = jnp.where(tgt[:, 1:2] == pcr, 1.0, 0.0)
    imgm = jnp.where(tgt[:, 0:1] == si, 1.0, 0.0)
    m = iou2 * clsm * imgm
    best = jnp.max(m, axis=0, keepdims=True)
    cor_ref[0] = jnp.where((best > iouv) & (keep > 0.0), 1.0, 0.0)


_NL = B * MAX_DET
_NP = NC * 10
_SHIFTS = (1, 2, 4, 8, 16, 32, 64, 128, 256, 512, 1024)
_EPS_DX = 1.4210854715202004e-14


def _ap_kernel(cor_ref, pcls_ref, valid_ref, clsl_ref, tcls_ref, x_ref,
               ap_ref, p_ref, r_ref, ngt_ref):
    cor = cor_ref[...]
    pclsv = pcls_ref[...]
    validv = valid_ref[...]
    clsl = clsl_ref[...]
    tclsv = tcls_ref[...]

    mask = jnp.where(pclsv == clsl, 1.0, 0.0) * validv
    tp = cor * mask
    fp = (1.0 - cor) * mask

    def csum(m):
        for s in _SHIFTS:
            m = m + jnp.concatenate(
                [jnp.zeros((s, _NP), jnp.float32), m[:-s, :]], axis=0)
        return m

    tpc = csum(tp)
    fpc = csum(fp)
    ngt = jnp.sum(jnp.where(tclsv == clsl, 1.0, 0.0), axis=0, keepdims=True)
    recall = tpc / (ngt + 1e-9)
    precision = tpc / (tpc + fpc + 1e-9)

    one = jnp.ones((1, _NP), jnp.float32)
    zero = jnp.zeros((1, _NP), jnp.float32)
    mrec = jnp.concatenate([zero, recall, one], axis=0)
    mpre = jnp.concatenate([one, precision, zero], axis=0)
    for s in _SHIFTS:
        mpre = jnp.maximum(mpre, jnp.concatenate(
            [mpre[s:, :], jnp.zeros((s, _NP), jnp.float32)], axis=0))

    sub2 = jax.lax.broadcasted_iota(jnp.int32, (_NL + 2, _NP), 0)

    def yat(xq):
        cnt = jnp.sum(jnp.where(mrec <= xq, 1, 0), axis=0, keepdims=True)
        i = jnp.clip(cnt, 1, _NL + 1)
        ohi = sub2 == i
        ohm = sub2 == (i - 1)
        fpi = jnp.sum(jnp.where(ohi, mpre, 0.0), axis=0, keepdims=True)
        fpm = jnp.sum(jnp.where(ohm, mpre, 0.0), axis=0, keepdims=True)
        xpi = jnp.sum(jnp.where(ohi, mrec, 0.0), axis=0, keepdims=True)
        xpm = jnp.sum(jnp.where(ohm, mrec, 0.0), axis=0, keepdims=True)
        df = fpi - fpm
        dx = xpi - xpm
        delta = xq - xpm
        dx0 = jnp.abs(dx) <= _EPS_DX
        return jnp.where(dx0, fpm, fpm + (delta / jnp.where(dx0, 1.0, dx)) * df)

    y0 = yat(x_ref[0])

    def body(q, carry):
        prev, acc = carry
        xq = x_ref[q]
        xqm = x_ref[q - 1]
        y = yat(xq)
        return (y, acc + (y + prev) * 0.5 * (xq - xqm))

    _, ap = jax.lax.fori_loop(1, 101, body, (y0, jnp.zeros((1, _NP))))

    f1 = 2 * precision * recall / (precision + recall + 1e-9)
    fmax = jnp.max(f1, axis=0, keepdims=True)
    subn = jax.lax.broadcasted_iota(jnp.int32, (_NL, _NP), 0)
    bi = jnp.min(jnp.where(f1 == fmax, subn, _NL), axis=0,
                 keepdims=True)
    ohb = subn == bi
    ap_ref[...] = ap
    p_ref[...] = jnp.sum(jnp.where(ohb, precision, 0.0), axis=0, keepdims=True)
    r_ref[...] = jnp.sum(jnp.where(ohb, recall, 0.0), axis=0, keepdims=True)
    ngt_ref[...] = ngt


def kernel(imgs, targets, out, shapes):
    iouv = jnp.linspace(0.5, 0.95, 10)
    iouv_col = iouv.reshape(10, 1)

    outT = jnp.swapaxes(out, 1, 2)
    score, cls = pl.pallas_call(
        _score_kernel,
        grid=(B,),
        in_specs=[pl.BlockSpec((1, 5 + NC, N_PRED), lambda i: (i, 0, 0))],
        out_specs=[pl.BlockSpec((1, 1, N_PRED), lambda i: (i, 0, 0)),
                   pl.BlockSpec((1, 1, N_PRED), lambda i: (i, 0, 0))],
        out_shape=[jax.ShapeDtypeStruct((B, 1, N_PRED), jnp.float32),
                   jax.ShapeDtypeStruct((B, 1, N_PRED), jnp.float32)],
    )(outT)
    score = score[:, 0, :]
    cls = cls[:, 0, :]

    top_s, top_i = jax.lax.top_k(score, MAX_DET)
    pxywh = jnp.take_along_axis(out[:, :, :4], top_i[..., None], axis=1)
    pc = jnp.take_along_axis(cls, top_i, axis=1)
    pt = jnp.swapaxes(pxywh, 1, 2)

    keep, confo, cor = pl.pallas_call(
        _nms_kernel,
        grid=(B,),
        in_specs=[
            pl.BlockSpec((1, MAX_DET, 4), lambda i: (i, 0, 0)),
            pl.BlockSpec((1, 4, MAX_DET), lambda i: (i, 0, 0)),
            pl.BlockSpec((1, MAX_DET, 1), lambda i: (i, 0, 0)),
            pl.BlockSpec((1, 1, MAX_DET), lambda i: (i, 0, 0)),
            pl.BlockSpec((1, 1, MAX_DET), lambda i: (i, 0, 0)),
            pl.BlockSpec((NT, 6), lambda i: (0, 0)),
            pl.BlockSpec((10, 1), lambda i: (0, 0)),
        ],
        out_specs=[
            pl.BlockSpec((1, 1, MAX_DET), lambda i: (i, 0, 0)),
            pl.BlockSpec((1, 1, MAX_DET), lambda i: (i, 0, 0)),
            pl.BlockSpec((1, 10, MAX_DET), lambda i: (i, 0, 0)),
        ],
        out_shape=[
            jax.ShapeDtypeStruct((B, 1, MAX_DET), jnp.float32),
            jax.ShapeDtypeStruct((B, 1, MAX_DET), jnp.float32),
            jax.ShapeDtypeStruct((B, 10, MAX_DET), jnp.float32),
        ],
        scratch_shapes=[pltpu.VMEM((MAX_DET, MAX_DET), jnp.float32)],
    )(pxywh, pt, pc[..., None], pc[:, None, :], top_s[:, None, :],
      targets, iouv_col)

    conf = confo.reshape(_NL)
    pcls = pc.reshape(_NL)
    keepf = keep.reshape(_NL)
    correct = jnp.swapaxes(cor, 1, 2).reshape(_NL, 10)

    order = jnp.argsort(-conf)
    cor_s = jnp.tile(correct[order], (1, NC))
    pcls_s = pcls[order].reshape(_NL, 1)
    valid_s = keepf[order].reshape(_NL, 1)
    cls_lane = (jnp.arange(_NP) // 10).astype(jnp.float32).reshape(1, _NP)
    tcls_col = targets[:, 1].reshape(NT, 1)
    x101 = jnp.linspace(0.0, 1.0, 101)

    ap_l, p_l, r_l, ngt_l = pl.pallas_call(
        _ap_kernel,
        in_specs=[
            pl.BlockSpec((_NL, _NP), lambda: (0, 0)),
            pl.BlockSpec((_NL, 1), lambda: (0, 0)),
            pl.BlockSpec((_NL, 1), lambda: (0, 0)),
            pl.BlockSpec((1, _NP), lambda: (0, 0)),
            pl.BlockSpec((NT, 1), lambda: (0, 0)),
            pl.BlockSpec(memory_space=pltpu.SMEM),
        ],
        out_specs=[
            pl.BlockSpec((1, _NP), lambda: (0, 0)),
            pl.BlockSpec((1, _NP), lambda: (0, 0)),
            pl.BlockSpec((1, _NP), lambda: (0, 0)),
            pl.BlockSpec((1, _NP), lambda: (0, 0)),
        ],
        out_shape=[jax.ShapeDtypeStruct((1, _NP), jnp.float32)] * 4,
    )(cor_s, pcls_s, valid_s, cls_lane, tcls_col, x101)

    ap2 = ap_l.reshape(NC, 10)
    p = p_l.reshape(NC, 10)[:, 0]
    r = r_l.reshape(NC, 10)[:, 0]
    n_gt = ngt_l.reshape(NC, 10)[:, 0]
    ap50 = ap2[:, 0]
    ap = jnp.mean(ap2, axis=1)
    has = (n_gt > 0).astype(jnp.float32)
    denom = jnp.maximum(jnp.sum(has), 1.0)
    mp = jnp.sum(p * has) / denom
    mr = jnp.sum(r * has) / denom
    map50 = jnp.sum(ap50 * has) / denom
    mapv = jnp.sum(ap * has) / denom
    fi = 0.1 * map50 + 0.9 * mapv
    return jnp.stack([fi, mp, mr, map50, mapv])
